# baseline (speedup 1.0000x reference)
import functools

import jax
import jax.numpy as jnp
from jax import lax
from jax.experimental import pallas as pl
from jax.experimental.pallas import tpu as pltpu
from jax.experimental.pallas import tpu_sc as plsc

MAX_LEN = 1000
HIDDEN_DIM = 128

NW = 32
ROWS = 16 * 64 * 50
ROWS_PER_W = ROWS // NW
CHUNK = 80
NCHUNK = ROWS_PER_W // CHUNK
LANES = 16
VECS_PER_ROW = HIDDEN_DIM // LANES


def _sc_kernel(emb_hbm, pos_hbm, pe_hbm, out_hbm,
               idx_v, in_v, pe_v, sem_in, sem_pe, sem_out):
  wid = lax.axis_index("s") * 2 + lax.axis_index("c")
  pltpu.sync_copy(pos_hbm.at[wid], idx_v)

  def chunk_body(c, _):
    base = wid * ROWS_PER_W + c * CHUNK
    pltpu.async_copy(emb_hbm.at[pl.ds(base, CHUNK)], in_v, sem_in).wait()
    pltpu.async_copy(pe_hbm.at[idx_v.at[c]], in_v, sem_pe, add=True).wait()
    pltpu.async_copy(in_v, out_hbm.at[pl.ds(base, CHUNK)], sem_out).wait()
    return 0

  lax.fori_loop(0, NCHUNK, chunk_body, 0)


def kernel(input_emb, position, pe):
  B, N, L, D = input_emb.shape
  emb2d = input_emb.reshape(ROWS, D)
  pos2d = position.reshape(NW, NCHUNK, CHUNK).astype(jnp.int32)

  run = functools.partial(
      pl.kernel,
      mesh=plsc.VectorSubcoreMesh(core_axis_name="c", subcore_axis_name="s"),
      out_type=jax.ShapeDtypeStruct((ROWS, D), jnp.float32),
      scratch_types=[
          pltpu.VMEM((NCHUNK, CHUNK), jnp.int32),
          pltpu.VMEM((CHUNK, D), jnp.float32),
          pltpu.VMEM((CHUNK, D), jnp.float32),
          pltpu.SemaphoreType.DMA,
          pltpu.SemaphoreType.DMA,
          pltpu.SemaphoreType.DMA,
      ],
  )(_sc_kernel)

  out = run(emb2d, pos2d, pe)
  return out.reshape(B, N, L, D)


# speedup vs baseline: 1.1363x; 1.1363x over previous
import functools

import jax
import jax.numpy as jnp
from jax import lax
from jax.experimental import pallas as pl
from jax.experimental.pallas import tpu as pltpu
from jax.experimental.pallas import tpu_sc as plsc

MAX_LEN = 1000
HIDDEN_DIM = 128

NW = 32
ROWS = 16 * 64 * 50
ROWS_PER_W = ROWS // NW
CHUNK = 80
NCHUNK = ROWS_PER_W // CHUNK
NBUF = 4


def _sc_kernel(emb_hbm, pos_hbm, pe_hbm, out_hbm,
               idx_v, b0, b1, b2, b3, sem_in, sem_ga, sem_out):
  bufs = (b0, b1, b2, b3)
  wid = lax.axis_index("s") * 2 + lax.axis_index("c")
  row0 = wid * ROWS_PER_W
  pltpu.sync_copy(pos_hbm.at[wid], idx_v)

  in_d = [None] * NCHUNK
  ga_d = [None] * NCHUNK
  out_d = [None] * NCHUNK

  def start_in(c):
    in_d[c] = pltpu.async_copy(
        emb_hbm.at[pl.ds(row0 + c * CHUNK, CHUNK)], bufs[c % NBUF], sem_in)

  def start_ga(c):
    in_d[c].wait()
    ga_d[c] = pltpu.async_copy(
        pe_hbm.at[idx_v.at[c]], bufs[c % NBUF], sem_ga, add=True)

  def start_out(c):
    ga_d[c].wait()
    out_d[c] = pltpu.async_copy(
        bufs[c % NBUF], out_hbm.at[pl.ds(row0 + c * CHUNK, CHUNK)], sem_out)

  for c in range(NCHUNK):
    if c >= NBUF:
      out_d[c - NBUF].wait()
    start_in(c)
    if c >= 1:
      start_ga(c - 1)
    if c >= 2:
      start_out(c - 2)

  start_ga(NCHUNK - 1)
  start_out(NCHUNK - 2)
  start_out(NCHUNK - 1)
  for c in range(NCHUNK - NBUF, NCHUNK):
    out_d[c].wait()


def kernel(input_emb, position, pe):
  B, N, L, D = input_emb.shape
  emb2d = input_emb.reshape(ROWS, D)
  pos2d = position.reshape(NW, NCHUNK, CHUNK).astype(jnp.int32)

  run = functools.partial(
      pl.kernel,
      mesh=plsc.VectorSubcoreMesh(core_axis_name="c", subcore_axis_name="s"),
      out_type=jax.ShapeDtypeStruct((ROWS, D), jnp.float32),
      scratch_types=[
          pltpu.VMEM((NCHUNK, CHUNK), jnp.int32),
          pltpu.VMEM((CHUNK, D), jnp.float32),
          pltpu.VMEM((CHUNK, D), jnp.float32),
          pltpu.VMEM((CHUNK, D), jnp.float32),
          pltpu.VMEM((CHUNK, D), jnp.float32),
          pltpu.SemaphoreType.DMA,
          pltpu.SemaphoreType.DMA,
          pltpu.SemaphoreType.DMA,
      ],
  )(_sc_kernel)

  out = run(emb2d, pos2d, pe)
  return out.reshape(B, N, L, D)


# speedup vs baseline: 1.6286x; 1.4333x over previous
import functools

import jax
import jax.numpy as jnp
from jax import lax
from jax.experimental import pallas as pl
from jax.experimental.pallas import tpu as pltpu
from jax.experimental.pallas import tpu_sc as plsc

MAX_LEN = 1000
HIDDEN_DIM = 128

NW = 32
B, N, L, D = 16, 64, 50, 128
PANELS_PER_W = (B * N) // NW
N_PER_W = N // 2
NBUF = 4


def _sc_kernel(emb_hbm, pos_hbm, pe_hbm, out_hbm,
               idx_v, b0, b1, b2, b3, sem_in, sem_ga, sem_out):
  bufs = (b0, b1, b2, b3)
  wid = lax.axis_index("s") * 2 + lax.axis_index("c")
  bb = wid // 2
  n0 = (wid % 2) * N_PER_W
  pltpu.sync_copy(pos_hbm.at[bb, pl.ds(n0, N_PER_W)], idx_v)

  in_d = [None] * PANELS_PER_W
  ga_d = [None] * PANELS_PER_W
  out_d = [None] * PANELS_PER_W

  def start_in(c):
    in_d[c] = pltpu.async_copy(
        emb_hbm.at[bb, n0 + c], bufs[c % NBUF], sem_in)

  def start_ga(c):
    in_d[c].wait()
    ga_d[c] = pltpu.async_copy(
        pe_hbm.at[idx_v.at[c]], bufs[c % NBUF], sem_ga, add=True)

  def start_out(c):
    ga_d[c].wait()
    out_d[c] = pltpu.async_copy(
        bufs[c % NBUF], out_hbm.at[bb, n0 + c], sem_out)

  for c in range(PANELS_PER_W):
    if c >= NBUF:
      out_d[c - NBUF].wait()
    start_in(c)
    if c >= 1:
      start_ga(c - 1)
    if c >= 2:
      start_out(c - 2)

  start_ga(PANELS_PER_W - 1)
  start_out(PANELS_PER_W - 2)
  start_out(PANELS_PER_W - 1)
  for c in range(PANELS_PER_W - NBUF, PANELS_PER_W):
    out_d[c].wait()


def kernel(input_emb, position, pe):
  pos = position.astype(jnp.int32)

  run = functools.partial(
      pl.kernel,
      mesh=plsc.VectorSubcoreMesh(core_axis_name="c", subcore_axis_name="s"),
      out_type=jax.ShapeDtypeStruct((B, N, L, D), jnp.float32),
      scratch_types=[
          pltpu.VMEM((N_PER_W, L), jnp.int32),
          pltpu.VMEM((L, D), jnp.float32),
          pltpu.VMEM((L, D), jnp.float32),
          pltpu.VMEM((L, D), jnp.float32),
          pltpu.VMEM((L, D), jnp.float32),
          pltpu.SemaphoreType.DMA,
          pltpu.SemaphoreType.DMA,
          pltpu.SemaphoreType.DMA,
      ],
  )(_sc_kernel)

  return run(input_emb, pos, pe)


# speedup vs baseline: 1.6301x; 1.0009x over previous
import functools

import jax
import jax.numpy as jnp
from jax import lax
from jax.experimental import pallas as pl
from jax.experimental.pallas import tpu as pltpu
from jax.experimental.pallas import tpu_sc as plsc

MAX_LEN = 1000
HIDDEN_DIM = 128

NW = 32
B, N, L, D = 16, 64, 50, 128
PANELS_PER_W = (B * N) // NW
N_PER_W = N // 2
NBUF = 4


def _sc_kernel(emb_hbm, pos_hbm, pe_hbm, out_hbm,
               idx_v, b0, b1, b2, b3, sem_in, sem_ga, sem_out):
  bufs = (b0, b1, b2, b3)
  wid = lax.axis_index("s") * 2 + lax.axis_index("c")
  bb = wid // 2
  n0 = (wid % 2) * N_PER_W
  pltpu.sync_copy(pos_hbm.at[bb, pl.ds(n0, N_PER_W)], idx_v)

  in_d = [None] * PANELS_PER_W
  ga_d = [None] * PANELS_PER_W
  out_d = [None] * PANELS_PER_W

  def start_in(c):
    in_d[c] = pltpu.async_copy(
        emb_hbm.at[bb, n0 + c], bufs[c % NBUF], sem_in)

  def start_ga(c):
    in_d[c].wait()
    ga_d[c] = pltpu.async_copy(
        pe_hbm.at[idx_v.at[c]], bufs[c % NBUF], sem_ga, add=True)

  def start_out(c):
    ga_d[c].wait()
    out_d[c] = pltpu.async_copy(
        bufs[c % NBUF], out_hbm.at[bb, n0 + c], sem_out)

  for c in range(PANELS_PER_W):
    if c >= NBUF:
      out_d[c - NBUF].wait()
    start_in(c)
    if c >= 1:
      start_ga(c - 1)
    if c >= 2:
      start_out(c - 2)

  start_ga(PANELS_PER_W - 1)
  start_out(PANELS_PER_W - 2)
  start_out(PANELS_PER_W - 1)
  for c in range(PANELS_PER_W - NBUF, PANELS_PER_W):
    out_d[c].wait()


def kernel(input_emb, position, pe):
  pos = position.astype(jnp.int32)

  run = functools.partial(
      pl.kernel,
      mesh=plsc.VectorSubcoreMesh(core_axis_name="c", subcore_axis_name="s"),
      compiler_params=pltpu.CompilerParams(use_tc_tiling_on_sc=True),
      out_type=jax.ShapeDtypeStruct((B, N, L, D), jnp.float32),
      scratch_types=[
          pltpu.VMEM((N_PER_W, L), jnp.int32),
          pltpu.VMEM((L, D), jnp.float32),
          pltpu.VMEM((L, D), jnp.float32),
          pltpu.VMEM((L, D), jnp.float32),
          pltpu.VMEM((L, D), jnp.float32),
          pltpu.SemaphoreType.DMA,
          pltpu.SemaphoreType.DMA,
          pltpu.SemaphoreType.DMA,
      ],
  )(_sc_kernel)

  return run(input_emb, pos, pe)


# speedup vs baseline: 3.1522x; 1.9338x over previous
import functools

import jax
import jax.numpy as jnp
from jax import lax
from jax.experimental import pallas as pl
from jax.experimental.pallas import tpu as pltpu
from jax.experimental.pallas import tpu_sc as plsc

MAX_LEN = 1000
HIDDEN_DIM = 128

NW = 32
B, N, L, D = 16, 64, 50, 128
PANELS = B * L
PANELS_PER_W = PANELS // NW
L_PER_W = L // 2
NBUF = 4


def _sc_kernel(emb_hbm, pos_hbm, pe_hbm, out_hbm,
               idx_v, b0, b1, b2, b3, sem_in, sem_ga, sem_out):
  bufs = (b0, b1, b2, b3)
  wid = lax.axis_index("s") * 2 + lax.axis_index("c")
  bb = wid // 2
  l0 = (wid % 2) * L_PER_W
  pltpu.sync_copy(pos_hbm.at[pl.ds(l0, L_PER_W), bb], idx_v)

  in_d = [None] * PANELS_PER_W
  ga_d = [None] * PANELS_PER_W
  out_d = [None] * PANELS_PER_W

  def start_in(c):
    in_d[c] = pltpu.async_copy(
        emb_hbm.at[bb, l0 + c], bufs[c % NBUF], sem_in)

  def start_ga(c):
    in_d[c].wait()
    ga_d[c] = pltpu.async_copy(
        pe_hbm.at[idx_v.at[c]], bufs[c % NBUF], sem_ga, add=True)

  def start_out(c):
    ga_d[c].wait()
    out_d[c] = pltpu.async_copy(
        bufs[c % NBUF], out_hbm.at[bb, l0 + c], sem_out)

  for c in range(PANELS_PER_W):
    if c >= NBUF:
      out_d[c - NBUF].wait()
    start_in(c)
    if c >= 1:
      start_ga(c - 1)
    if c >= 2:
      start_out(c - 2)

  start_ga(PANELS_PER_W - 1)
  start_out(PANELS_PER_W - 2)
  start_out(PANELS_PER_W - 1)
  for c in range(PANELS_PER_W - NBUF, PANELS_PER_W):
    out_d[c].wait()


def kernel(input_emb, position, pe):
  emb_t = jnp.transpose(input_emb, (0, 2, 1, 3))
  pos_t = jnp.transpose(position.astype(jnp.int32), (2, 0, 1))

  run = functools.partial(
      pl.kernel,
      mesh=plsc.VectorSubcoreMesh(core_axis_name="c", subcore_axis_name="s"),
      out_type=jax.ShapeDtypeStruct((B, L, N, D), jnp.float32),
      scratch_types=[
          pltpu.VMEM((PANELS_PER_W, N), jnp.int32),
          pltpu.VMEM((N, D), jnp.float32),
          pltpu.VMEM((N, D), jnp.float32),
          pltpu.VMEM((N, D), jnp.float32),
          pltpu.VMEM((N, D), jnp.float32),
          pltpu.SemaphoreType.DMA,
          pltpu.SemaphoreType.DMA,
          pltpu.SemaphoreType.DMA,
      ],
  )(_sc_kernel)

  out_t = run(emb_t, pos_t, pe)
  return jnp.transpose(out_t, (0, 2, 1, 3))


# speedup vs baseline: 3.1880x; 1.0114x over previous
import functools

import jax
import jax.numpy as jnp
from jax import lax
from jax.experimental import pallas as pl
from jax.experimental.pallas import tpu as pltpu
from jax.experimental.pallas import tpu_sc as plsc

MAX_LEN = 1000
HIDDEN_DIM = 128

NW = 32
B, N, L, D = 16, 64, 50, 128
PANELS = B * L
PANELS_PER_W = PANELS // NW
L_PER_W = L // 2
NBUF = 8
GA_SKEW = 3
OUT_SKEW = 6


def _sc_kernel(emb_hbm, pos_hbm, pe_hbm, out_hbm,
               idx_v, b0, b1, b2, b3, b4, b5, b6, b7, sem_in, sem_ga, sem_out):
  bufs = (b0, b1, b2, b3, b4, b5, b6, b7)
  wid = lax.axis_index("s") * 2 + lax.axis_index("c")
  bb = wid // 2
  l0 = (wid % 2) * L_PER_W
  pltpu.sync_copy(pos_hbm.at[pl.ds(l0, L_PER_W), bb], idx_v)

  in_d = [None] * PANELS_PER_W
  ga_d = [None] * PANELS_PER_W
  out_d = [None] * PANELS_PER_W

  def start_in(c):
    in_d[c] = pltpu.async_copy(
        emb_hbm.at[bb, l0 + c], bufs[c % NBUF], sem_in)

  def start_ga(c):
    in_d[c].wait()
    ga_d[c] = pltpu.async_copy(
        pe_hbm.at[idx_v.at[c]], bufs[c % NBUF], sem_ga, add=True)

  def start_out(c):
    ga_d[c].wait()
    out_d[c] = pltpu.async_copy(
        bufs[c % NBUF], out_hbm.at[bb, l0 + c], sem_out)

  for c in range(PANELS_PER_W):
    if c >= NBUF:
      out_d[c - NBUF].wait()
    start_in(c)
    if c >= GA_SKEW:
      start_ga(c - GA_SKEW)
    if c >= OUT_SKEW:
      start_out(c - OUT_SKEW)

  for c in range(PANELS_PER_W - GA_SKEW, PANELS_PER_W):
    start_ga(c)
  for c in range(PANELS_PER_W - OUT_SKEW, PANELS_PER_W):
    start_out(c)
  for c in range(max(0, PANELS_PER_W - NBUF), PANELS_PER_W):
    out_d[c].wait()


def kernel(input_emb, position, pe):
  emb_t = jnp.transpose(input_emb, (0, 2, 1, 3))
  pos_t = jnp.transpose(position.astype(jnp.int32), (2, 0, 1))

  run = functools.partial(
      pl.kernel,
      mesh=plsc.VectorSubcoreMesh(core_axis_name="c", subcore_axis_name="s"),
      out_type=jax.ShapeDtypeStruct((B, L, N, D), jnp.float32),
      scratch_types=[
          pltpu.VMEM((PANELS_PER_W, N), jnp.int32),
          pltpu.VMEM((N, D), jnp.float32),
          pltpu.VMEM((N, D), jnp.float32),
          pltpu.VMEM((N, D), jnp.float32),
          pltpu.VMEM((N, D), jnp.float32),
          pltpu.VMEM((N, D), jnp.float32),
          pltpu.VMEM((N, D), jnp.float32),
          pltpu.VMEM((N, D), jnp.float32),
          pltpu.VMEM((N, D), jnp.float32),
          pltpu.SemaphoreType.DMA,
          pltpu.SemaphoreType.DMA,
          pltpu.SemaphoreType.DMA,
      ],
  )(_sc_kernel)

  out_t = run(emb_t, pos_t, pe)
  return jnp.transpose(out_t, (0, 2, 1, 3))


# speedup vs baseline: 4.3397x; 1.3613x over previous
import functools

import jax
import jax.numpy as jnp
from jax import lax
from jax.experimental import pallas as pl
from jax.experimental.pallas import tpu as pltpu
from jax.experimental.pallas import tpu_sc as plsc

MAX_LEN = 1000
HIDDEN_DIM = 128

NW = 32
B, N, L, D = 16, 64, 50, 128
PANELS = B * L
PANELS_PER_W = PANELS // NW
L_PER_W = L // 2
NBUF = 8
GA_SKEW = 3
OUT_SKEW = 6


def _sc_kernel(emb_hbm, pos_hbm, pe_hbm, out_hbm,
               idx_v, pe_sh, b0, b1, b2, b3, b4, b5, b6, b7,
               sem_in, sem_ga, sem_out):
  bufs = (b0, b1, b2, b3, b4, b5, b6, b7)
  sid = lax.axis_index("s")
  wid = sid * 2 + lax.axis_index("c")
  bb = wid // 2
  l0 = (wid % 2) * L_PER_W

  @pl.when(sid == 0)
  def _stage():
    pltpu.sync_copy(pe_hbm, pe_sh)

  pltpu.sync_copy(pos_hbm.at[pl.ds(l0, L_PER_W), bb], idx_v)
  plsc.subcore_barrier()

  in_d = [None] * PANELS_PER_W
  ga_d = [None] * PANELS_PER_W
  out_d = [None] * PANELS_PER_W

  def start_in(c):
    in_d[c] = pltpu.async_copy(
        emb_hbm.at[bb, l0 + c], bufs[c % NBUF], sem_in)

  def start_ga(c):
    in_d[c].wait()
    ga_d[c] = pltpu.async_copy(
        pe_sh.at[idx_v.at[c]], bufs[c % NBUF], sem_ga, add=True)

  def start_out(c):
    ga_d[c].wait()
    out_d[c] = pltpu.async_copy(
        bufs[c % NBUF], out_hbm.at[bb, l0 + c], sem_out)

  for c in range(PANELS_PER_W):
    if c >= NBUF:
      out_d[c - NBUF].wait()
    start_in(c)
    if c >= GA_SKEW:
      start_ga(c - GA_SKEW)
    if c >= OUT_SKEW:
      start_out(c - OUT_SKEW)

  for c in range(PANELS_PER_W - GA_SKEW, PANELS_PER_W):
    start_ga(c)
  for c in range(PANELS_PER_W - OUT_SKEW, PANELS_PER_W):
    start_out(c)
  for c in range(max(0, PANELS_PER_W - NBUF), PANELS_PER_W):
    out_d[c].wait()


def kernel(input_emb, position, pe):
  emb_t = jnp.transpose(input_emb, (0, 2, 1, 3))
  pos_t = jnp.transpose(position.astype(jnp.int32), (2, 0, 1))

  run = functools.partial(
      pl.kernel,
      mesh=plsc.VectorSubcoreMesh(core_axis_name="c", subcore_axis_name="s"),
      out_type=jax.ShapeDtypeStruct((B, L, N, D), jnp.float32),
      scratch_types=[
          pltpu.VMEM((PANELS_PER_W, N), jnp.int32),
          pltpu.VMEM_SHARED((MAX_LEN, D), jnp.float32),
          pltpu.VMEM((N, D), jnp.float32),
          pltpu.VMEM((N, D), jnp.float32),
          pltpu.VMEM((N, D), jnp.float32),
          pltpu.VMEM((N, D), jnp.float32),
          pltpu.VMEM((N, D), jnp.float32),
          pltpu.VMEM((N, D), jnp.float32),
          pltpu.VMEM((N, D), jnp.float32),
          pltpu.VMEM((N, D), jnp.float32),
          pltpu.SemaphoreType.DMA,
          pltpu.SemaphoreType.DMA,
          pltpu.SemaphoreType.DMA,
      ],
  )(_sc_kernel)

  out_t = run(emb_t, pos_t, pe)
  return jnp.transpose(out_t, (0, 2, 1, 3))


# speedup vs baseline: 4.4865x; 1.0338x over previous
import functools

import jax
import jax.numpy as jnp
from jax import lax
from jax.experimental import pallas as pl
from jax.experimental.pallas import tpu as pltpu
from jax.experimental.pallas import tpu_sc as plsc

MAX_LEN = 1000
HIDDEN_DIM = 128

NW = 32
B, N, L, D = 16, 64, 50, 128
PANELS = B * L
PANELS_PER_W = PANELS // NW
L_PER_W = L // 2
CH_SIZES = [2] * 12 + [1]
CH_OFFS = [sum(CH_SIZES[:i]) for i in range(len(CH_SIZES))]
NCH = len(CH_SIZES)
NBUF = 6
GA_SKEW = 2
OUT_SKEW = 4


def _sc_kernel(emb_hbm, pos_hbm, pe_hbm, out_hbm,
               idx_v, pe_sh, b0, b1, b2, b3, b4, b5,
               sem_in, sem_ga, sem_out):
  bufs = (b0, b1, b2, b3, b4, b5)
  sid = lax.axis_index("s")
  wid = sid * 2 + lax.axis_index("c")
  bb = wid // 2
  l0 = (wid % 2) * L_PER_W

  @pl.when(sid == 0)
  def _stage():
    pltpu.sync_copy(pe_hbm, pe_sh)

  pltpu.sync_copy(pos_hbm.at[pl.ds(l0, L_PER_W), bb], idx_v)
  plsc.subcore_barrier()

  in_d = [None] * NCH
  ga_d = [None] * NCH
  out_d = [None] * NCH

  def start_in(c):
    sz = CH_SIZES[c]
    in_d[c] = pltpu.async_copy(
        emb_hbm.at[bb, pl.ds(l0 + CH_OFFS[c], sz)],
        bufs[c % NBUF].at[pl.ds(0, sz)], sem_in)

  def start_ga(c):
    in_d[c].wait()
    ga_d[c] = [
        pltpu.async_copy(
            pe_sh.at[idx_v.at[CH_OFFS[c] + k]], bufs[c % NBUF].at[k],
            sem_ga, add=True)
        for k in range(CH_SIZES[c])
    ]

  def start_out(c):
    for d in ga_d[c]:
      d.wait()
    sz = CH_SIZES[c]
    out_d[c] = pltpu.async_copy(
        bufs[c % NBUF].at[pl.ds(0, sz)],
        out_hbm.at[bb, pl.ds(l0 + CH_OFFS[c], sz)], sem_out)

  for c in range(NCH):
    if c >= NBUF:
      out_d[c - NBUF].wait()
    start_in(c)
    if c >= GA_SKEW:
      start_ga(c - GA_SKEW)
    if c >= OUT_SKEW:
      start_out(c - OUT_SKEW)

  for c in range(NCH - GA_SKEW, NCH):
    start_ga(c)
  for c in range(NCH - OUT_SKEW, NCH):
    start_out(c)
  for c in range(max(0, NCH - NBUF), NCH):
    out_d[c].wait()


def kernel(input_emb, position, pe):
  emb_t = jnp.transpose(input_emb, (0, 2, 1, 3))
  pos_t = jnp.transpose(position.astype(jnp.int32), (2, 0, 1))

  run = functools.partial(
      pl.kernel,
      mesh=plsc.VectorSubcoreMesh(core_axis_name="c", subcore_axis_name="s"),
      out_type=jax.ShapeDtypeStruct((B, L, N, D), jnp.float32),
      scratch_types=[
          pltpu.VMEM((PANELS_PER_W, N), jnp.int32),
          pltpu.VMEM_SHARED((MAX_LEN, D), jnp.float32),
          pltpu.VMEM((2, N, D), jnp.float32),
          pltpu.VMEM((2, N, D), jnp.float32),
          pltpu.VMEM((2, N, D), jnp.float32),
          pltpu.VMEM((2, N, D), jnp.float32),
          pltpu.VMEM((2, N, D), jnp.float32),
          pltpu.VMEM((2, N, D), jnp.float32),
          pltpu.SemaphoreType.DMA,
          pltpu.SemaphoreType.DMA,
          pltpu.SemaphoreType.DMA,
      ],
  )(_sc_kernel)

  out_t = run(emb_t, pos_t, pe)
  return jnp.transpose(out_t, (0, 2, 1, 3))


# speedup vs baseline: 4.5004x; 1.0031x over previous
import functools

import jax
import jax.numpy as jnp
from jax import lax
from jax.experimental import pallas as pl
from jax.experimental.pallas import tpu as pltpu
from jax.experimental.pallas import tpu_sc as plsc

MAX_LEN = 1000
HIDDEN_DIM = 128

NW = 32
B, N, L, D = 16, 64, 50, 128
PANELS = B * L
PANELS_PER_W = PANELS // NW
L_PER_W = L // 2
CH_SIZES = [2] * 12 + [1]
CH_OFFS = [sum(CH_SIZES[:i]) for i in range(len(CH_SIZES))]
NCH = len(CH_SIZES)
NBUF = 6
GA_SKEW = 2
OUT_SKEW = 4


def _sc_kernel(emb_hbm, pos_hbm, pe_hbm, out_hbm,
               idx_v, pe_sh, b0, b1, b2, b3, b4, b5,
               sem_in, sem_ga, sem_out):
  bufs = (b0, b1, b2, b3, b4, b5)
  sid = lax.axis_index("s")
  wid = sid * 2 + lax.axis_index("c")
  bb = wid // 2
  l0 = (wid % 2) * L_PER_W

  @pl.when(sid == 0)
  def _stage():
    pltpu.sync_copy(pe_hbm, pe_sh)

  pltpu.sync_copy(pos_hbm.at[pl.ds(l0, L_PER_W), bb], idx_v)
  plsc.subcore_barrier()

  in_d = [None] * NCH
  ga_d = [None] * NCH
  out_d = [None] * NCH

  def start_in(c):
    sz = CH_SIZES[c]
    in_d[c] = pltpu.async_copy(
        emb_hbm.at[bb, pl.ds(l0 + CH_OFFS[c], sz)],
        bufs[c % NBUF].at[pl.ds(0, sz)], sem_in)

  def start_ga(c):
    in_d[c].wait()
    ga_d[c] = [
        pltpu.async_copy(
            pe_sh.at[idx_v.at[CH_OFFS[c] + k]], bufs[c % NBUF].at[k],
            sem_ga, add=True)
        for k in range(CH_SIZES[c])
    ]

  def start_out(c):
    for d in ga_d[c]:
      d.wait()
    sz = CH_SIZES[c]
    out_d[c] = pltpu.async_copy(
        bufs[c % NBUF].at[pl.ds(0, sz)],
        out_hbm.at[bb, pl.ds(l0 + CH_OFFS[c], sz)], sem_out)

  for c in range(NCH):
    if c >= NBUF:
      out_d[c - NBUF].wait()
    start_in(c)
    if c >= GA_SKEW:
      start_ga(c - GA_SKEW)
    if c >= OUT_SKEW:
      start_out(c - OUT_SKEW)

  for c in range(NCH - GA_SKEW, NCH):
    start_ga(c)
  for c in range(NCH - OUT_SKEW, NCH):
    start_out(c)
  for c in range(max(0, NCH - NBUF), NCH):
    out_d[c].wait()


def kernel(input_emb, position, pe):
  emb_t = jnp.transpose(input_emb, (0, 2, 1, 3))
  pos_t = jnp.transpose(position.astype(jnp.int32), (2, 0, 1))

  run = functools.partial(
      pl.kernel,
      mesh=plsc.VectorSubcoreMesh(core_axis_name="c", subcore_axis_name="s"),
      compiler_params=pltpu.CompilerParams(skip_device_barrier=True),
      out_type=jax.ShapeDtypeStruct((B, L, N, D), jnp.float32),
      scratch_types=[
          pltpu.VMEM((PANELS_PER_W, N), jnp.int32),
          pltpu.VMEM_SHARED((MAX_LEN, D), jnp.float32),
          pltpu.VMEM((2, N, D), jnp.float32),
          pltpu.VMEM((2, N, D), jnp.float32),
          pltpu.VMEM((2, N, D), jnp.float32),
          pltpu.VMEM((2, N, D), jnp.float32),
          pltpu.VMEM((2, N, D), jnp.float32),
          pltpu.VMEM((2, N, D), jnp.float32),
          pltpu.SemaphoreType.DMA,
          pltpu.SemaphoreType.DMA,
          pltpu.SemaphoreType.DMA,
      ],
  )(_sc_kernel)

  out_t = run(emb_t, pos_t, pe)
  return jnp.transpose(out_t, (0, 2, 1, 3))
